# baseline (speedup 1.0000x reference)
import jax
import jax.numpy as jnp
from jax.experimental import pallas as pl
from jax.experimental.pallas import tpu as pltpu

_DU = 20
_DZ = 6
_DZU = _DU * _DZ
_BT = 128


def _mlp_chain(h, ws):
    n = len(ws)
    for i, (wt, b) in enumerate(ws):
        h = jnp.dot(h, wt, precision=jax.lax.Precision.HIGHEST,
                    preferred_element_type=jnp.float32) + b
        if i < n - 1:
            h = jnp.maximum(h, 0.0)
    return h


def _body(x_ref,
          w1a, b1a, w2a, b2a, w3a, b3a, w4a, b4a,
          w1b, b1b, w2b, b2b, w3b, b3b, w4b, b4b,
          f1_ref, g1_ref, f2_ref, g2_ref):
    bt = x_ref.shape[0]
    x = x_ref[...]
    xm = jnp.concatenate([x[:, -1:], x[:, :-1]], axis=1)
    xp = jnp.concatenate([x[:, 1:], x[:, :1]], axis=1)

    def first_layer(w, b):
        h = (xm[:, :, None] * w[0][None, None, :]
             + x[:, :, None] * w[1][None, None, :]
             + xp[:, :, None] * w[2][None, None, :]) + b[...][None, :, :]
        return jnp.maximum(h, 0.0).reshape(bt * _DU, 16)

    h1 = first_layer(w1a[...], b1a)
    h2 = first_layer(w1b[...], b1b)

    out1 = _mlp_chain(h1, [(w2a[...], b2a[...]), (w3a[...], b3a[...]),
                           (w4a[...], b4a[...])])
    out2 = _mlp_chain(h2, [(w2b[...], b2b[...]), (w3b[...], b3b[...]),
                           (w4b[...], b4b[...])])

    out1 = out1.reshape(bt, _DU, 1 + 3 * _DZ)
    out2 = out2.reshape(bt, _DU, _DZ + 5 * _DZ * _DZ)

    f1_ref[...] = out1[:, :, 0]
    f2_ref[...] = out2[:, :, :_DZ].reshape(bt, _DZU)

    def cat(parts, axis):
        parts = [p for p in parts if p.shape[axis] > 0]
        return parts[0] if len(parts) == 1 else jnp.concatenate(parts, axis)

    for i in range(_DU):
        vals = out1[:, i, 1:]
        o = ((i - 1) * _DZ) % _DZU
        if o + 3 * _DZ <= _DZU:
            row = cat([jnp.zeros((bt, o), jnp.float32), vals,
                       jnp.zeros((bt, _DZU - o - 3 * _DZ), jnp.float32)], 1)
        else:
            w1 = _DZU - o
            row = cat([vals[:, w1:], jnp.zeros((bt, _DZU - 3 * _DZ), jnp.float32),
                       vals[:, :w1]], 1)
        g1_ref[:, i, :] = row

    for jb in range(_DU):
        o = ((jb - 2) * _DZ) % _DZU
        for s in range(_DZ):
            vals = out2[:, jb, _DZ + 30 * s:_DZ + 30 * (s + 1)]
            if o + 5 * _DZ <= _DZU:
                row = cat([jnp.zeros((bt, o), jnp.float32), vals,
                           jnp.zeros((bt, _DZU - o - 5 * _DZ), jnp.float32)], 1)
            else:
                w1 = _DZU - o
                row = cat([vals[:, w1:],
                           jnp.zeros((bt, _DZU - 5 * _DZ), jnp.float32),
                           vals[:, :w1]], 1)
            g2_ref[:, 6 * jb + s, :] = row


def kernel(x, params1, params2):
    B = x.shape[0]
    bt = _BT if B % _BT == 0 else B
    grid = (B // bt,)

    wargs = []
    for (W, b) in list(params1) + list(params2):
        wargs.append(W.T)
        wargs.append(b.reshape(1, -1))

    x_spec = pl.BlockSpec((bt, _DU), lambda i: (i, 0))
    w_specs = [pl.BlockSpec(w.shape, lambda i: (0, 0)) for w in wargs]
    out_specs = [
        pl.BlockSpec((bt, _DU), lambda i: (i, 0)),
        pl.BlockSpec((bt, _DU, _DZU), lambda i: (i, 0, 0)),
        pl.BlockSpec((bt, _DZU), lambda i: (i, 0)),
        pl.BlockSpec((bt, _DZU, _DZU), lambda i: (i, 0, 0)),
    ]
    out_shape = [
        jax.ShapeDtypeStruct((B, _DU), jnp.float32),
        jax.ShapeDtypeStruct((B, _DU, _DZU), jnp.float32),
        jax.ShapeDtypeStruct((B, _DZU), jnp.float32),
        jax.ShapeDtypeStruct((B, _DZU, _DZU), jnp.float32),
    ]

    f1, g1, f2, g2 = pl.pallas_call(
        _body,
        grid=grid,
        in_specs=[x_spec] + w_specs,
        out_specs=out_specs,
        out_shape=out_shape,
        compiler_params=pltpu.CompilerParams(
            dimension_semantics=("parallel",),
        ),
    )(x, *wargs)

    return f1[..., None], g1, f2[..., None], g2
